# baseline (speedup 1.0000x reference)
import functools

import jax
import jax.numpy as jnp
from jax import lax
from jax.experimental import pallas as pl
from jax.experimental.pallas import tpu as pltpu
from jax.experimental.shard_map import shard_map
from jax.sharding import Mesh, PartitionSpec as P


def _round_up(x, m):
    return ((x + m - 1) // m) * m


def _hard_mine_kernel(xrows_ref, xall_ref, sq_row_ref, sq_col_ref,
                      t_col_ref, t_row_ref, hinge_ref, corr_ref,
                      *, n, bn, margin, mask_rows):
    xb = xrows_ref[...]
    xall = xall_ref[...]

    gram = lax.dot_general(xb, xall, (((1,), (1,)), ((), ())),
                           preferred_element_type=jnp.float32)

    part = sq_col_ref[...] - 2.0 * gram
    same_id = t_col_ref[...] == t_row_ref[...]
    ap = jnp.max(jnp.where(same_id, part, -jnp.inf), axis=1, keepdims=True)
    an = jnp.min(jnp.where(same_id, jnp.inf, part), axis=1, keepdims=True)

    sq_row = sq_row_ref[...]
    dist_ap = jnp.sqrt(jnp.maximum(ap + sq_row, 1e-12))
    dist_an = jnp.sqrt(jnp.maximum(an + sq_row, 1e-12))

    hinge = jnp.maximum(dist_ap - dist_an + margin, 0.0)
    corr = (dist_an >= dist_ap).astype(jnp.int32)

    if mask_rows:
        row_ids = pl.program_id(0) * bn + lax.broadcasted_iota(
            jnp.int32, (bn, 1), 0)
        row_valid = row_ids < n
        hinge = jnp.where(row_valid, hinge, 0.0)
        corr = jnp.where(row_valid, corr, 0)

    hinge_ref[...] = hinge
    corr_ref[...] = corr


def _triplet_call(n_rows, n_valid_rows, n_pad, d_pad, bn, margin):
    body = functools.partial(
        _hard_mine_kernel, n=n_valid_rows, bn=bn, margin=float(margin),
        mask_rows=bool(n_rows != n_valid_rows))

    def resident(shape):
        return pl.BlockSpec(shape, lambda i: tuple(0 for _ in shape),
                            pipeline_mode=pl.Buffered(1))

    return pl.pallas_call(
        body,
        grid=(n_rows // bn,),
        in_specs=[
            pl.BlockSpec((bn, d_pad), lambda i: (i, 0)),
            resident((n_pad, d_pad)),
            pl.BlockSpec((bn, 1), lambda i: (i, 0)),
            resident((1, n_pad)),
            pl.BlockSpec((bn, 1), lambda i: (i, 0)),
            resident((1, n_pad)),
        ],
        out_specs=[
            pl.BlockSpec((bn, 1), lambda i: (i, 0)),
            pl.BlockSpec((bn, 1), lambda i: (i, 0)),
        ],
        out_shape=[
            jax.ShapeDtypeStruct((n_rows, 1), jnp.float32),
            jax.ShapeDtypeStruct((n_rows, 1), jnp.int32),
        ],
        compiler_params=pltpu.CompilerParams(
            dimension_semantics=("parallel",),
            vmem_limit_bytes=64 * 1024 * 1024),
    )


def kernel(inputs, targets):
    margin = 0.3
    x = jnp.asarray(inputs, jnp.float32)
    t = jnp.asarray(targets, jnp.int32)
    n, d = x.shape

    bn = 256 if n >= 256 else _round_up(min(n, 128), 8)
    n_pad = _round_up(n, bn)
    d_pad = _round_up(d, 128)

    if (n_pad, d_pad) != (n, d):
        xp = jnp.zeros((n_pad, d_pad), jnp.float32).at[:n, :d].set(x)
    else:
        xp = x

    sq = jnp.sum(xp * xp, axis=1)
    if n_pad != n:
        col_valid = jnp.arange(n_pad) < n
        sq_col = jnp.where(col_valid, sq, jnp.float32(1e30)).reshape(1, n_pad)
        sentinel = jnp.min(t) - jnp.int32(1)
        tp = jnp.full((n_pad,), sentinel, jnp.int32).at[:n].set(t)
    else:
        sq_col = sq.reshape(1, n_pad)
        tp = t
    sq_row = sq.reshape(n_pad, 1)
    t_col = tp.reshape(n_pad, 1)
    t_row = tp.reshape(1, n_pad)

    x16 = xp.astype(jnp.bfloat16)

    devices = jax.devices()
    n_blocks = n_pad // bn
    n_dev = 2 if (len(devices) >= 2 and n_blocks % 2 == 0) else 1

    if n_dev == 1:
        hinge, corr = _triplet_call(n_pad, n, n_pad, d_pad, bn, margin)(
            x16, x16, sq_row, sq_col, t_col, t_row)
    else:
        mesh = Mesh(devices[:n_dev], ("dp",))
        half = n_pad // n_dev

        def _shard(x_rows, x_full, sqr, sqc, tc, tr):
            call = _triplet_call(half, n if n_pad != n else half,
                                 n_pad, d_pad, bn, margin)
            if n_pad != n:
                idx = lax.axis_index("dp")
                h_loc, c_loc = call(x_rows, x_full, sqr, sqc, tc, tr)
                row_ids = idx * half + lax.broadcasted_iota(
                    jnp.int32, (half, 1), 0)
                valid = row_ids < n
                return (jnp.where(valid, h_loc, 0.0),
                        jnp.where(valid, c_loc, 0))
            return call(x_rows, x_full, sqr, sqc, tc, tr)

        hinge, corr = shard_map(
            _shard, mesh=mesh,
            in_specs=(P("dp"), P(), P("dp"), P(), P("dp"), P()),
            out_specs=(P("dp"), P("dp")),
            check_rep=False,
        )(x16, x16, sq_row, sq_col, t_col, t_row)

    loss = jnp.sum(hinge) / jnp.float32(n)
    correct = jnp.sum(corr)
    return loss, correct


# speedup vs baseline: 7.2890x; 7.2890x over previous
import functools

import jax
import jax.numpy as jnp
from jax import lax
from jax.experimental import pallas as pl
from jax.experimental.pallas import tpu as pltpu


def _round_up(x, m):
    return ((x + m - 1) // m) * m


def _tile_ij(g, nt):
    i = jnp.int32(0)
    thr = 0
    for r in range(nt - 1):
        thr += nt - r
        i = i + (g >= thr).astype(jnp.int32)
    off = i * nt - (i * (i - 1)) // 2
    j = g - off + i
    return i, j


def _tri_kernel(x_ref, sq_col_ref, sq_row_ref, t_row_ref, t_col_ref,
                loss_ref, corr_ref, ap_acc, an_acc,
                apb_acc, anb_acc, *, n, bt, nt, margin):
    g = pl.program_id(0)

    i, j = _tile_ij(g, nt)
    ri = i * bt
    rj = j * bt

    @pl.when(g == 0)
    def _init():
        ap_acc[...] = jnp.full((ap_acc.shape), -jnp.inf, jnp.float32)
        an_acc[...] = jnp.full((an_acc.shape), jnp.inf, jnp.float32)
        apb_acc[...] = jnp.full((apb_acc.shape), -jnp.inf, jnp.float32)
        anb_acc[...] = jnp.full((anb_acc.shape), jnp.inf, jnp.float32)

    xi = x_ref[pl.ds(ri, bt), :]
    xj = x_ref[pl.ds(rj, bt), :]
    gram = lax.dot_general(xi, xj, (((1,), (1,)), ((), ())),
                           preferred_element_type=jnp.float32)

    u = (sq_row_ref[pl.ds(ri, bt), :]
         + sq_col_ref[:, pl.ds(rj, bt)]) - 2.0 * gram
    same = t_col_ref[pl.ds(ri, bt), :] == t_row_ref[:, pl.ds(rj, bt)]
    u_pos = jnp.where(same, u, -jnp.inf)
    u_neg = jnp.where(same, jnp.inf, u)

    ap_acc[pl.ds(ri, bt), :] = jnp.maximum(
        ap_acc[pl.ds(ri, bt), :], jnp.max(u_pos, axis=1, keepdims=True))
    an_acc[pl.ds(ri, bt), :] = jnp.minimum(
        an_acc[pl.ds(ri, bt), :], jnp.min(u_neg, axis=1, keepdims=True))
    apb_acc[:, pl.ds(rj, bt)] = jnp.maximum(
        apb_acc[:, pl.ds(rj, bt)], jnp.max(u_pos, axis=0, keepdims=True))
    anb_acc[:, pl.ds(rj, bt)] = jnp.minimum(
        anb_acc[:, pl.ds(rj, bt)], jnp.min(u_neg, axis=0, keepdims=True))

    @pl.when(g == pl.num_programs(0) - 1)
    def _finalize():
        ap_t = jnp.transpose(ap_acc[...])
        an_t = jnp.transpose(an_acc[...])
        ap = jnp.maximum(ap_t, apb_acc[...])
        an = jnp.minimum(an_t, anb_acc[...])
        dist_ap = jnp.sqrt(jnp.maximum(ap, 1e-12))
        dist_an = jnp.sqrt(jnp.maximum(an, 1e-12))
        hinge = jnp.maximum(dist_ap - dist_an + margin, 0.0)
        corr = (dist_an >= dist_ap).astype(jnp.int32)
        loss_ref[...] = (jnp.sum(hinge, keepdims=True)
                         / jnp.float32(n)).reshape(1, 1)
        corr_ref[...] = jnp.sum(corr, keepdims=True).reshape(1, 1)


def _tri_call(n, d, bt, margin):
    nt = n // bt
    body = functools.partial(_tri_kernel, n=n, bt=bt, nt=nt,
                             margin=float(margin))

    def resident(shape):
        return pl.BlockSpec(shape, lambda g: tuple(0 for _ in shape),
                            pipeline_mode=pl.Buffered(1))

    return pl.pallas_call(
        body,
        grid=(nt * (nt + 1) // 2,),
        in_specs=[
            resident((n, d)),
            resident((1, n)),
            resident((n, 1)),
            resident((1, n)),
            resident((n, 1)),
        ],
        out_specs=[
            pl.BlockSpec((1, 1), lambda g: (0, 0)),
            pl.BlockSpec((1, 1), lambda g: (0, 0)),
        ],
        out_shape=[
            jax.ShapeDtypeStruct((1, 1), jnp.float32),
            jax.ShapeDtypeStruct((1, 1), jnp.int32),
        ],
        scratch_shapes=[
            pltpu.VMEM((n, 1), jnp.float32),
            pltpu.VMEM((n, 1), jnp.float32),
            pltpu.VMEM((1, n), jnp.float32),
            pltpu.VMEM((1, n), jnp.float32),
        ],
        compiler_params=pltpu.CompilerParams(
            dimension_semantics=("arbitrary",),
            vmem_limit_bytes=64 * 1024 * 1024),
    )


def _full_kernel(xrows_ref, xall_ref, sq_row_ref, sq_col_ref,
                 t_col_ref, t_row_ref, hinge_ref, corr_ref,
                 *, n, bn, margin, mask_rows):
    xb = xrows_ref[...]
    xall = xall_ref[...]
    gram = lax.dot_general(xb, xall, (((1,), (1,)), ((), ())),
                           preferred_element_type=jnp.float32)
    part = sq_col_ref[...] - 2.0 * gram
    same_id = t_col_ref[...] == t_row_ref[...]
    ap = jnp.max(jnp.where(same_id, part, -jnp.inf), axis=1, keepdims=True)
    an = jnp.min(jnp.where(same_id, jnp.inf, part), axis=1, keepdims=True)
    sq_row = sq_row_ref[...]
    dist_ap = jnp.sqrt(jnp.maximum(ap + sq_row, 1e-12))
    dist_an = jnp.sqrt(jnp.maximum(an + sq_row, 1e-12))
    hinge = jnp.maximum(dist_ap - dist_an + margin, 0.0)
    corr = (dist_an >= dist_ap).astype(jnp.int32)
    if mask_rows:
        row_ids = pl.program_id(0) * bn + lax.broadcasted_iota(
            jnp.int32, (bn, 1), 0)
        row_valid = row_ids < n
        hinge = jnp.where(row_valid, hinge, 0.0)
        corr = jnp.where(row_valid, corr, 0)
    hinge_ref[...] = hinge
    corr_ref[...] = corr


def _full_call(n, n_pad, d_pad, bn, margin):
    body = functools.partial(_full_kernel, n=n, bn=bn, margin=float(margin),
                             mask_rows=bool(n_pad != n))

    def resident(shape):
        return pl.BlockSpec(shape, lambda i: tuple(0 for _ in shape),
                            pipeline_mode=pl.Buffered(1))

    return pl.pallas_call(
        body,
        grid=(n_pad // bn,),
        in_specs=[
            pl.BlockSpec((bn, d_pad), lambda i: (i, 0)),
            resident((n_pad, d_pad)),
            pl.BlockSpec((bn, 1), lambda i: (i, 0)),
            resident((1, n_pad)),
            pl.BlockSpec((bn, 1), lambda i: (i, 0)),
            resident((1, n_pad)),
        ],
        out_specs=[
            pl.BlockSpec((bn, 1), lambda i: (i, 0)),
            pl.BlockSpec((bn, 1), lambda i: (i, 0)),
        ],
        out_shape=[
            jax.ShapeDtypeStruct((n_pad, 1), jnp.float32),
            jax.ShapeDtypeStruct((n_pad, 1), jnp.int32),
        ],
        compiler_params=pltpu.CompilerParams(
            dimension_semantics=("parallel",),
            vmem_limit_bytes=64 * 1024 * 1024),
    )


def kernel(inputs, targets):
    margin = 0.3
    x = jnp.asarray(inputs, jnp.float32)
    t = jnp.asarray(targets, jnp.int32)
    n, d = x.shape

    bt = 1024
    if n % bt == 0 and d % 128 == 0:
        sq = jnp.sum(x * x, axis=1)
        x16 = x.astype(jnp.bfloat16)
        loss2d, corr2d = _tri_call(n, d, bt, margin)(
            x16, sq.reshape(1, n), sq.reshape(n, 1),
            t.reshape(1, n), t.reshape(n, 1))
        return loss2d[0, 0], corr2d[0, 0]

    bn = 256 if n >= 256 else _round_up(min(n, 128), 8)
    n_pad = _round_up(n, bn)
    d_pad = _round_up(d, 128)
    if (n_pad, d_pad) != (n, d):
        xp = jnp.zeros((n_pad, d_pad), jnp.float32).at[:n, :d].set(x)
    else:
        xp = x
    sq = jnp.sum(xp * xp, axis=1)
    if n_pad != n:
        col_valid = jnp.arange(n_pad) < n
        sq_col = jnp.where(col_valid, sq, jnp.float32(1e30)).reshape(1, n_pad)
        sentinel = jnp.min(t) - jnp.int32(1)
        tp = jnp.full((n_pad,), sentinel, jnp.int32).at[:n].set(t)
    else:
        sq_col = sq.reshape(1, n_pad)
        tp = t
    x16 = xp.astype(jnp.bfloat16)
    hinge, corr = _full_call(n, n_pad, d_pad, bn, margin)(
        x16, x16, sq.reshape(n_pad, 1), sq_col,
        tp.reshape(n_pad, 1), tp.reshape(1, n_pad))
    loss = jnp.sum(hinge) / jnp.float32(n)
    correct = jnp.sum(corr)
    return loss, correct


# speedup vs baseline: 7.7516x; 1.0635x over previous
import functools

import jax
import jax.numpy as jnp
from jax import lax
from jax.experimental import pallas as pl
from jax.experimental.pallas import tpu as pltpu


def _round_up(x, m):
    return ((x + m - 1) // m) * m


def _tile_ij(g, nt):
    i = jnp.int32(0)
    thr = 0
    for r in range(nt - 1):
        thr += nt - r
        i = i + (g >= thr).astype(jnp.int32)
    off = i * nt - (i * (i - 1)) // 2
    j = g - off + i
    return i, j


def _tri_kernel(x_ref, sq_col_ref, t_row_ref,
                loss_ref, corr_ref, ap_acc, an_acc,
                apb_acc, anb_acc, sq_row_s, t_col_s, *, n, bt, nt, margin):
    g = pl.program_id(0)

    i, j = _tile_ij(g, nt)
    ri = i * bt
    rj = j * bt

    @pl.when(g == 0)
    def _init():
        sq_row_s[...] = jnp.transpose(sq_col_ref[...])
        t_col_s[...] = jnp.transpose(t_row_ref[...])
        ap_acc[...] = jnp.full((ap_acc.shape), -jnp.inf, jnp.float32)
        an_acc[...] = jnp.full((an_acc.shape), jnp.inf, jnp.float32)
        apb_acc[...] = jnp.full((apb_acc.shape), -jnp.inf, jnp.float32)
        anb_acc[...] = jnp.full((anb_acc.shape), jnp.inf, jnp.float32)

    xi = x_ref[pl.ds(ri, bt), :]
    xj = x_ref[pl.ds(rj, bt), :]
    gram = lax.dot_general(xi, xj, (((1,), (1,)), ((), ())),
                           preferred_element_type=jnp.float32)

    u = (sq_row_s[pl.ds(ri, bt), :]
         + sq_col_ref[:, pl.ds(rj, bt)]) - 2.0 * gram
    same = t_col_s[pl.ds(ri, bt), :] == t_row_ref[:, pl.ds(rj, bt)]
    u_pos = jnp.where(same, u, -jnp.inf)
    u_neg = jnp.where(same, jnp.inf, u)

    ap_acc[pl.ds(ri, bt), :] = jnp.maximum(
        ap_acc[pl.ds(ri, bt), :], jnp.max(u_pos, axis=1, keepdims=True))
    an_acc[pl.ds(ri, bt), :] = jnp.minimum(
        an_acc[pl.ds(ri, bt), :], jnp.min(u_neg, axis=1, keepdims=True))
    apb_acc[:, pl.ds(rj, bt)] = jnp.maximum(
        apb_acc[:, pl.ds(rj, bt)], jnp.max(u_pos, axis=0, keepdims=True))
    anb_acc[:, pl.ds(rj, bt)] = jnp.minimum(
        anb_acc[:, pl.ds(rj, bt)], jnp.min(u_neg, axis=0, keepdims=True))

    @pl.when(g == pl.num_programs(0) - 1)
    def _finalize():
        ap_t = jnp.transpose(ap_acc[...])
        an_t = jnp.transpose(an_acc[...])
        ap = jnp.maximum(ap_t, apb_acc[...])
        an = jnp.minimum(an_t, anb_acc[...])
        dist_ap = jnp.sqrt(jnp.maximum(ap, 1e-12))
        dist_an = jnp.sqrt(jnp.maximum(an, 1e-12))
        hinge = jnp.maximum(dist_ap - dist_an + margin, 0.0)
        corr = (dist_an >= dist_ap).astype(jnp.int32)
        loss_ref[...] = (jnp.sum(hinge, keepdims=True)
                         / jnp.float32(n)).reshape(1, 1)
        corr_ref[...] = jnp.sum(corr, keepdims=True).reshape(1, 1)


def _tri_call(n, d, bt, margin):
    nt = n // bt
    body = functools.partial(_tri_kernel, n=n, bt=bt, nt=nt,
                             margin=float(margin))

    def resident(shape):
        return pl.BlockSpec(shape, lambda g: tuple(0 for _ in shape),
                            pipeline_mode=pl.Buffered(1))

    return pl.pallas_call(
        body,
        grid=(nt * (nt + 1) // 2,),
        in_specs=[
            resident((n, d)),
            resident((1, n)),
            resident((1, n)),
        ],
        out_specs=[
            pl.BlockSpec((1, 1), lambda g: (0, 0)),
            pl.BlockSpec((1, 1), lambda g: (0, 0)),
        ],
        out_shape=[
            jax.ShapeDtypeStruct((1, 1), jnp.float32),
            jax.ShapeDtypeStruct((1, 1), jnp.int32),
        ],
        scratch_shapes=[
            pltpu.VMEM((n, 1), jnp.float32),
            pltpu.VMEM((n, 1), jnp.float32),
            pltpu.VMEM((1, n), jnp.float32),
            pltpu.VMEM((1, n), jnp.float32),
            pltpu.VMEM((n, 1), jnp.float32),
            pltpu.VMEM((n, 1), jnp.int32),
        ],
        compiler_params=pltpu.CompilerParams(
            dimension_semantics=("arbitrary",),
            vmem_limit_bytes=64 * 1024 * 1024),
    )


def _full_kernel(xrows_ref, xall_ref, sq_row_ref, sq_col_ref,
                 t_col_ref, t_row_ref, hinge_ref, corr_ref,
                 *, n, bn, margin, mask_rows):
    xb = xrows_ref[...]
    xall = xall_ref[...]
    gram = lax.dot_general(xb, xall, (((1,), (1,)), ((), ())),
                           preferred_element_type=jnp.float32)
    part = sq_col_ref[...] - 2.0 * gram
    same_id = t_col_ref[...] == t_row_ref[...]
    ap = jnp.max(jnp.where(same_id, part, -jnp.inf), axis=1, keepdims=True)
    an = jnp.min(jnp.where(same_id, jnp.inf, part), axis=1, keepdims=True)
    sq_row = sq_row_ref[...]
    dist_ap = jnp.sqrt(jnp.maximum(ap + sq_row, 1e-12))
    dist_an = jnp.sqrt(jnp.maximum(an + sq_row, 1e-12))
    hinge = jnp.maximum(dist_ap - dist_an + margin, 0.0)
    corr = (dist_an >= dist_ap).astype(jnp.int32)
    if mask_rows:
        row_ids = pl.program_id(0) * bn + lax.broadcasted_iota(
            jnp.int32, (bn, 1), 0)
        row_valid = row_ids < n
        hinge = jnp.where(row_valid, hinge, 0.0)
        corr = jnp.where(row_valid, corr, 0)
    hinge_ref[...] = hinge
    corr_ref[...] = corr


def _full_call(n, n_pad, d_pad, bn, margin):
    body = functools.partial(_full_kernel, n=n, bn=bn, margin=float(margin),
                             mask_rows=bool(n_pad != n))

    def resident(shape):
        return pl.BlockSpec(shape, lambda i: tuple(0 for _ in shape),
                            pipeline_mode=pl.Buffered(1))

    return pl.pallas_call(
        body,
        grid=(n_pad // bn,),
        in_specs=[
            pl.BlockSpec((bn, d_pad), lambda i: (i, 0)),
            resident((n_pad, d_pad)),
            pl.BlockSpec((bn, 1), lambda i: (i, 0)),
            resident((1, n_pad)),
            pl.BlockSpec((bn, 1), lambda i: (i, 0)),
            resident((1, n_pad)),
        ],
        out_specs=[
            pl.BlockSpec((bn, 1), lambda i: (i, 0)),
            pl.BlockSpec((bn, 1), lambda i: (i, 0)),
        ],
        out_shape=[
            jax.ShapeDtypeStruct((n_pad, 1), jnp.float32),
            jax.ShapeDtypeStruct((n_pad, 1), jnp.int32),
        ],
        compiler_params=pltpu.CompilerParams(
            dimension_semantics=("parallel",),
            vmem_limit_bytes=64 * 1024 * 1024),
    )


def kernel(inputs, targets):
    margin = 0.3
    x = jnp.asarray(inputs, jnp.float32)
    t = jnp.asarray(targets, jnp.int32)
    n, d = x.shape

    bt = 1024
    if n % bt == 0 and d % 128 == 0:
        sq = jnp.sum(x * x, axis=1)
        x16 = x.astype(jnp.bfloat16)
        loss2d, corr2d = _tri_call(n, d, bt, margin)(
            x16, sq.reshape(1, n), t.reshape(1, n))
        return loss2d[0, 0], corr2d[0, 0]

    bn = 256 if n >= 256 else _round_up(min(n, 128), 8)
    n_pad = _round_up(n, bn)
    d_pad = _round_up(d, 128)
    if (n_pad, d_pad) != (n, d):
        xp = jnp.zeros((n_pad, d_pad), jnp.float32).at[:n, :d].set(x)
    else:
        xp = x
    sq = jnp.sum(xp * xp, axis=1)
    if n_pad != n:
        col_valid = jnp.arange(n_pad) < n
        sq_col = jnp.where(col_valid, sq, jnp.float32(1e30)).reshape(1, n_pad)
        sentinel = jnp.min(t) - jnp.int32(1)
        tp = jnp.full((n_pad,), sentinel, jnp.int32).at[:n].set(t)
    else:
        sq_col = sq.reshape(1, n_pad)
        tp = t
    x16 = xp.astype(jnp.bfloat16)
    hinge, corr = _full_call(n, n_pad, d_pad, bn, margin)(
        x16, x16, sq.reshape(n_pad, 1), sq_col,
        tp.reshape(n_pad, 1), tp.reshape(1, n_pad))
    loss = jnp.sum(hinge) / jnp.float32(n)
    correct = jnp.sum(corr)
    return loss, correct
